# baseline (speedup 1.0000x reference)
import jax
import jax.numpy as jnp
from jax import lax
from jax.experimental import pallas as pl
from jax.experimental.pallas import tpu as pltpu
from jax.experimental.pallas import tpu_sc as plsc

_B = 16384
_F = 64
_NC = 2
_NS = 16
_L = 16
_NW = _NC * _NS
_BPW = _B // _NW
_NCH = 4
_CROWS = _BPW // _NCH
_CGRP = _CROWS // _L
_NCK = _F // _L


def _sc_body(u_hbm, i_hbm, idx_hbm, w_hbm, pop_hbm, out_hbm,
             u0, u1, i0, i1, idx_v, pops_v, w_v, out_v, du_s, di_s,
             sem, gsem):
    wid = lax.axis_index("s") * _NC + lax.axis_index("c")
    base = wid * _BPW

    pltpu.sync_copy(idx_hbm.at[pl.ds(base, _BPW)], idx_v)
    c_pop = pltpu.async_copy(pop_hbm.at[idx_v], pops_v, gsem)
    c_w = pltpu.async_copy(w_hbm, w_v, sem)

    ubufs = [u0, u1]
    ibufs = [i0, i1]

    def start_chunk(c):
        rows = pl.ds(base + c * _CROWS, _CROWS)
        cu = pltpu.async_copy(u_hbm.at[rows, :], ubufs[c % 2], sem)
        ci = pltpu.async_copy(i_hbm.at[rows, :], ibufs[c % 2], sem)
        return cu, ci

    inflight = [start_chunk(0), start_chunk(1)]

    c_w.wait()
    c_pop.wait()

    lane = lax.iota(jnp.int32, _L)
    m_first = lane == 0
    rot_idx = [(lane + k) % _L for k in (8, 4, 2, 1)]
    wu = [w_v[pl.ds(c * _L, _L)] for c in range(_NCK)]
    wi = [w_v[pl.ds(_F + c * _L, _L)] for c in range(_NCK)]
    icpt = w_v[pl.ds(2 * _F, _L)]

    for c in range(_NCH):
        cu, ci = inflight[c % 2]
        cu.wait()
        ci.wait()
        u_v = ubufs[c % 2]
        i_v = ibufs[c % 2]
        crow0 = c * _CROWS

        def grp_body(g, carry, u_v=u_v, i_v=i_v, crow0=crow0):
            row0 = g * _L
            for b in range(_L // 4):
                vals = []
                for j4 in range(4):
                    r = row0 + b * 4 + j4
                    pu = u_v[r, pl.ds(0, _L)] * wu[0]
                    pi = i_v[r, pl.ds(0, _L)] * wi[0]
                    for k in range(1, _NCK):
                        pu = pu + u_v[r, pl.ds(k * _L, _L)] * wu[k]
                        pi = pi + i_v[r, pl.ds(k * _L, _L)] * wi[k]
                    vals.extend((pu, pi))
                for ridx in rot_idx:
                    vals = [p + p.at[ridx].get(mode="promise_in_bounds")
                            for p in vals]
                for j4 in range(4):
                    j = b * 4 + j4
                    plsc.store_compressed(du_s.at[pl.ds(j, _L)],
                                          vals[2 * j4], mask=m_first)
                    plsc.store_compressed(di_s.at[pl.ds(j, _L)],
                                          vals[2 * j4 + 1], mask=m_first)
            du = du_s[pl.ds(0, _L)]
            di = di_s[pl.ds(0, _L)]
            su = 1.0 / (1.0 + jnp.exp(-du))
            si = 1.0 / (1.0 + jnp.exp(-di))
            p = su * si
            score = jnp.where(p > 0.0, p, jnp.exp(p) - 1.0) + 1.0
            pops = pops_v[pl.ds(crow0 + row0, _L)]
            bits = plsc.bitcast(pops, jnp.int32)
            y = plsc.bitcast(jnp.int32(0x5F3759DF) - (bits >> 1), jnp.float32)
            y = y * (1.5 - 0.5 * pops * y * y)
            y = y * (1.5 - 0.5 * pops * y * y)
            y = y * (1.5 - 0.5 * pops * y * y)
            sqrt_pops = pops * y
            out_v[pl.ds(crow0 + row0, _L)] = score * sqrt_pops + icpt
            return carry

        lax.fori_loop(0, _CGRP, grp_body, 0)
        if c + 2 < _NCH:
            inflight[c % 2] = start_chunk(c + 2)

    pltpu.sync_copy(out_v, out_hbm.at[pl.ds(base, _BPW)])


_sc_call = pl.kernel(
    _sc_body,
    out_type=jax.ShapeDtypeStruct((_B,), jnp.float32),
    mesh=plsc.VectorSubcoreMesh(core_axis_name="c", subcore_axis_name="s"),
    compiler_params=pltpu.CompilerParams(needs_layout_passes=False, use_tc_tiling_on_sc=True),
    scratch_types=[
        pltpu.VMEM((_CROWS, _F), jnp.float32),
        pltpu.VMEM((_CROWS, _F), jnp.float32),
        pltpu.VMEM((_CROWS, _F), jnp.float32),
        pltpu.VMEM((_CROWS, _F), jnp.float32),
        pltpu.VMEM((_BPW,), jnp.int32),
        pltpu.VMEM((_BPW,), jnp.float32),
        pltpu.VMEM((2 * _F + _L,), jnp.float32),
        pltpu.VMEM((_BPW,), jnp.float32),
        pltpu.VMEM((2 * _L,), jnp.float32),
        pltpu.VMEM((2 * _L,), jnp.float32),
        pltpu.SemaphoreType.DMA,
        pltpu.SemaphoreType.DMA,
    ],
)


@jax.jit
def kernel(users, items, item_pop_idx, W_user, W_item, intercept, popularity):
    params = jnp.concatenate([
        W_user.astype(jnp.float32).reshape(_F),
        W_item.astype(jnp.float32).reshape(_F),
        jnp.broadcast_to(intercept.astype(jnp.float32), (_L,)),
    ])
    idx = item_pop_idx.astype(jnp.int32)
    return _sc_call(users, items, idx, params, popularity)


# speedup vs baseline: 1.4324x; 1.4324x over previous
import jax
import jax.numpy as jnp
from jax import lax
from jax.experimental import pallas as pl
from jax.experimental.pallas import tpu as pltpu
from jax.experimental.pallas import tpu_sc as plsc

_B = 16384
_F = 64
_NC = 2
_NS = 16
_L = 16
_NW = _NC * _NS
_BPW = _B // _NW
_NCH = 4
_CCOLS = _BPW // _NCH
_CGRP = _CCOLS // _L


def _sc_body(ut_hbm, it_hbm, idx_hbm, w_hbm, pop_hbm, out_hbm,
             u0, u1, i0, i1, idx_v, pops_v, w_v, out_v, sem, gsem):
    wid = lax.axis_index("s") * _NC + lax.axis_index("c")
    base = wid * _BPW

    pltpu.sync_copy(idx_hbm.at[pl.ds(base, _BPW)], idx_v)
    c_pop = pltpu.async_copy(pop_hbm.at[idx_v], pops_v, gsem)
    c_w = pltpu.async_copy(w_hbm, w_v, sem)

    ubufs = [u0, u1]
    ibufs = [i0, i1]

    def start_chunk(c):
        cols = pl.ds(base + c * _CCOLS, _CCOLS)
        cu = pltpu.async_copy(ut_hbm.at[:, cols], ubufs[c % 2], sem)
        ci = pltpu.async_copy(it_hbm.at[:, cols], ibufs[c % 2], sem)
        return cu, ci

    inflight = [start_chunk(0), start_chunk(1)]

    c_w.wait()
    c_pop.wait()

    icpt = w_v[pl.ds(2 * _F * _L, _L)]
    zero = jnp.zeros((_L,), jnp.float32)

    for c in range(_NCH):
        cu, ci = inflight[c % 2]
        cu.wait()
        ci.wait()
        u_v = ubufs[c % 2]
        i_v = ibufs[c % 2]
        ccol0 = c * _CCOLS

        def f_body(f, carry, u_v=u_v, i_v=i_v):
            accs_u, accs_i = carry
            wu_f = w_v[pl.ds(f * _L, _L)]
            wi_f = w_v[pl.ds(_F * _L + f * _L, _L)]
            new_u = tuple(
                accs_u[g] + u_v[f, pl.ds(g * _L, _L)] * wu_f
                for g in range(_CGRP))
            new_i = tuple(
                accs_i[g] + i_v[f, pl.ds(g * _L, _L)] * wi_f
                for g in range(_CGRP))
            return new_u, new_i

        init = (tuple(zero for _ in range(_CGRP)),
                tuple(zero for _ in range(_CGRP)))
        accs_u, accs_i = lax.fori_loop(0, _F, f_body, init)

        for g in range(_CGRP):
            su = 1.0 / (1.0 + jnp.exp(-accs_u[g]))
            si = 1.0 / (1.0 + jnp.exp(-accs_i[g]))
            p = su * si
            score = jnp.where(p > 0.0, p, jnp.exp(p) - 1.0) + 1.0
            off = ccol0 + g * _L
            pops = pops_v[pl.ds(off, _L)]
            bits = plsc.bitcast(pops, jnp.int32)
            y = plsc.bitcast(jnp.int32(0x5F3759DF) - (bits >> 1), jnp.float32)
            y = y * (1.5 - 0.5 * pops * y * y)
            y = y * (1.5 - 0.5 * pops * y * y)
            y = y * (1.5 - 0.5 * pops * y * y)
            sqrt_pops = pops * y
            out_v[pl.ds(off, _L)] = score * sqrt_pops + icpt

        if c + 2 < _NCH:
            inflight[c % 2] = start_chunk(c + 2)

    pltpu.sync_copy(out_v, out_hbm.at[pl.ds(base, _BPW)])


_sc_call = pl.kernel(
    _sc_body,
    out_type=jax.ShapeDtypeStruct((_B,), jnp.float32),
    mesh=plsc.VectorSubcoreMesh(core_axis_name="c", subcore_axis_name="s"),
    compiler_params=pltpu.CompilerParams(needs_layout_passes=False),
    scratch_types=[
        pltpu.VMEM((_F, _CCOLS), jnp.float32),
        pltpu.VMEM((_F, _CCOLS), jnp.float32),
        pltpu.VMEM((_F, _CCOLS), jnp.float32),
        pltpu.VMEM((_F, _CCOLS), jnp.float32),
        pltpu.VMEM((_BPW,), jnp.int32),
        pltpu.VMEM((_BPW,), jnp.float32),
        pltpu.VMEM((2 * _F * _L + _L,), jnp.float32),
        pltpu.VMEM((_BPW,), jnp.float32),
        pltpu.SemaphoreType.DMA,
        pltpu.SemaphoreType.DMA,
    ],
)


@jax.jit
def kernel(users, items, item_pop_idx, W_user, W_item, intercept, popularity):
    params = jnp.concatenate([
        jnp.broadcast_to(W_user.astype(jnp.float32), (_F, _L)).reshape(_F * _L),
        jnp.broadcast_to(W_item.astype(jnp.float32), (_F, _L)).reshape(_F * _L),
        jnp.broadcast_to(intercept.astype(jnp.float32), (_L,)),
    ])
    idx = item_pop_idx.astype(jnp.int32)
    return _sc_call(users.T, items.T, idx, params, popularity)


# speedup vs baseline: 1.5163x; 1.0586x over previous
import jax
import jax.numpy as jnp
from jax import lax
from jax.experimental import pallas as pl
from jax.experimental.pallas import tpu as pltpu
from jax.experimental.pallas import tpu_sc as plsc

_B = 16384
_F = 64
_NC = 2
_NS = 16
_L = 16
_NW = _NC * _NS
_BPW = _B // _NW
_NCH = 4
_CCOLS = _BPW // _NCH
_CGRP = _CCOLS // _L


def _sc_body(ut_hbm, it_hbm, idx_hbm, w_hbm, pop_hbm, out_hbm,
             u0, u1, i0, i1, idx_v, pops_v, w_v, out_v, sem, gsem):
    wid = lax.axis_index("s") * _NC + lax.axis_index("c")
    base = wid * _BPW

    pltpu.sync_copy(idx_hbm.at[pl.ds(base, _BPW)], idx_v)
    c_pop = pltpu.async_copy(pop_hbm.at[idx_v], pops_v, gsem)
    c_w = pltpu.async_copy(w_hbm, w_v, sem)

    ubufs = [u0, u1]
    ibufs = [i0, i1]

    def start_chunk(c):
        cols = pl.ds(base + c * _CCOLS, _CCOLS)
        cu = pltpu.async_copy(ut_hbm.at[:, cols], ubufs[c % 2], sem)
        ci = pltpu.async_copy(it_hbm.at[:, cols], ibufs[c % 2], sem)
        return cu, ci

    inflight = [start_chunk(0), start_chunk(1)]

    c_w.wait()
    c_pop.wait()

    icpt = w_v[pl.ds(2 * _F * _L, _L)]
    zero = jnp.zeros((_L,), jnp.float32)

    for c in range(_NCH):
        cu, ci = inflight[c % 2]
        cu.wait()
        ci.wait()
        u_v = ubufs[c % 2]
        i_v = ibufs[c % 2]
        ccol0 = c * _CCOLS

        def f_body(f, carry, u_v=u_v, i_v=i_v):
            accs_u, accs_i = carry
            wu_f = w_v[pl.ds(f * _L, _L)]
            wi_f = w_v[pl.ds(_F * _L + f * _L, _L)]
            new_u = tuple(
                accs_u[g] + u_v[f, pl.ds(g * _L, _L)] * wu_f
                for g in range(_CGRP))
            new_i = tuple(
                accs_i[g] + i_v[f, pl.ds(g * _L, _L)] * wi_f
                for g in range(_CGRP))
            return new_u, new_i

        init = (tuple(zero for _ in range(_CGRP)),
                tuple(zero for _ in range(_CGRP)))
        accs_u, accs_i = lax.fori_loop(0, _F, f_body, init)

        eu = [jnp.exp(-accs_u[g]) for g in range(_CGRP)]
        ei = [jnp.exp(-accs_i[g]) for g in range(_CGRP)]
        den = [(1.0 + eu[g]) * (1.0 + ei[g]) for g in range(_CGRP)]
        pops = [pops_v[pl.ds(ccol0 + g * _L, _L)] for g in range(_CGRP)]
        ys = []
        for g in range(_CGRP):
            bits = plsc.bitcast(pops[g], jnp.int32)
            ys.append(plsc.bitcast(jnp.int32(0x5F3759DF) - (bits >> 1),
                                   jnp.float32))
        for _ in range(2):
            ys = [y * (1.5 - 0.5 * pops[g] * y * y)
                  for g, y in enumerate(ys)]
        for g in range(_CGRP):
            score = 1.0 / den[g] + 1.0
            sqrt_pops = pops[g] * ys[g]
            out_v[pl.ds(ccol0 + g * _L, _L)] = score * sqrt_pops + icpt

        if c + 2 < _NCH:
            inflight[c % 2] = start_chunk(c + 2)

    pltpu.sync_copy(out_v, out_hbm.at[pl.ds(base, _BPW)])


_sc_call = pl.kernel(
    _sc_body,
    out_type=jax.ShapeDtypeStruct((_B,), jnp.float32),
    mesh=plsc.VectorSubcoreMesh(core_axis_name="c", subcore_axis_name="s"),
    compiler_params=pltpu.CompilerParams(needs_layout_passes=False),
    scratch_types=[
        pltpu.VMEM((_F, _CCOLS), jnp.float32),
        pltpu.VMEM((_F, _CCOLS), jnp.float32),
        pltpu.VMEM((_F, _CCOLS), jnp.float32),
        pltpu.VMEM((_F, _CCOLS), jnp.float32),
        pltpu.VMEM((_BPW,), jnp.int32),
        pltpu.VMEM((_BPW,), jnp.float32),
        pltpu.VMEM((2 * _F * _L + _L,), jnp.float32),
        pltpu.VMEM((_BPW,), jnp.float32),
        pltpu.SemaphoreType.DMA,
        pltpu.SemaphoreType.DMA,
    ],
)


@jax.jit
def kernel(users, items, item_pop_idx, W_user, W_item, intercept, popularity):
    w_all = jnp.concatenate([W_user.astype(jnp.float32),
                             W_item.astype(jnp.float32)], axis=0)
    params = jnp.concatenate([
        jnp.broadcast_to(w_all, (2 * _F, _L)).reshape(2 * _F * _L),
        jnp.broadcast_to(intercept.astype(jnp.float32), (_L,)),
    ])
    idx = item_pop_idx.astype(jnp.int32)
    return _sc_call(users.T, items.T, idx, params, popularity)
